# baseline (speedup 1.0000x reference)
import jax
import jax.numpy as jnp
from jax.experimental import pallas as pl
from jax.experimental.pallas import tpu as pltpu

N_V = 2048
N_E = 4096
D = 256
TJ = 256


def _prep_body(hvt_ref, tt_ref, ef_ref, w_ref, p_ref, lsc_ref, ew_ref, hp_ref):
    i = pl.program_id(0)

    @pl.when(i == 0)
    def _():
        hp_ref[...] = jnp.dot(p_ref[...], hvt_ref[...],
                              preferred_element_type=jnp.float32)
        ew_ref[...] = jnp.dot(ef_ref[...], w_ref[...],
                              preferred_element_type=jnp.float32)

    lsc_ref[...] = tt_ref[...] * hp_ref[...]


def _main_body(lsc_ref, t_ref, adj_ref, ew_ref, bias_ref, out_ref, acc_ref):
    j = pl.program_id(0)
    nj = pl.num_programs(0)

    mult = jnp.dot(lsc_ref[...], t_ref[...],
                   preferred_element_type=jnp.float32)
    rows = jax.lax.broadcasted_iota(jnp.int32, (N_E, TJ), 0)
    cols = jax.lax.broadcasted_iota(jnp.int32, (N_E, TJ), 1) + j * TJ
    adj = adj_ref[...]
    a = jnp.where(rows == cols, adj, mult * adj)
    cmax = jnp.max(a, axis=0, keepdims=True)
    contrib = jnp.dot(a / cmax, ew_ref[...],
                      preferred_element_type=jnp.float32)

    @pl.when(j == 0)
    def _():
        acc_ref[...] = contrib

    @pl.when(j > 0)
    def _():
        acc_ref[...] += contrib

    @pl.when(j == nj - 1)
    def _():
        out_ref[...] = acc_ref[...] + bias_ref[...]


def kernel(H_v, edge_features, adj_e, adj_v, T, weight, bias, p):
    del adj_v
    t_t = T.T
    h_v_t = H_v.T

    n_prep = 4
    lsc, ew = pl.pallas_call(
        _prep_body,
        grid=(n_prep,),
        in_specs=[
            pl.BlockSpec((D, N_V), lambda i: (0, 0)),
            pl.BlockSpec((N_E // n_prep, N_V), lambda i: (i, 0)),
            pl.BlockSpec((N_E, D), lambda i: (0, 0)),
            pl.BlockSpec((D, D), lambda i: (0, 0)),
            pl.BlockSpec((1, D), lambda i: (0, 0)),
        ],
        out_specs=[
            pl.BlockSpec((N_E // n_prep, N_V), lambda i: (i, 0)),
            pl.BlockSpec((N_E, D), lambda i: (0, 0)),
        ],
        out_shape=[
            jax.ShapeDtypeStruct((N_E, N_V), jnp.float32),
            jax.ShapeDtypeStruct((N_E, D), jnp.float32),
        ],
        scratch_shapes=[pltpu.VMEM((1, N_V), jnp.float32)],
    )(h_v_t, t_t, edge_features, weight, p)

    out = pl.pallas_call(
        _main_body,
        grid=(N_E // TJ,),
        in_specs=[
            pl.BlockSpec((N_E, N_V), lambda j: (0, 0)),
            pl.BlockSpec((N_V, TJ), lambda j: (0, j)),
            pl.BlockSpec((N_E, TJ), lambda j: (0, j)),
            pl.BlockSpec((TJ, D), lambda j: (j, 0)),
            pl.BlockSpec((1, D), lambda j: (0, 0)),
        ],
        out_specs=pl.BlockSpec((N_E, D), lambda j: (0, 0)),
        out_shape=jax.ShapeDtypeStruct((N_E, D), jnp.float32),
        scratch_shapes=[pltpu.VMEM((N_E, D), jnp.float32)],
        compiler_params=pltpu.CompilerParams(
            vmem_limit_bytes=100 * 1024 * 1024),
    )(lsc, T, adj_e, ew, bias.reshape(1, D))

    return out


# speedup vs baseline: 1.0488x; 1.0488x over previous
import jax
import jax.numpy as jnp
from jax.experimental import pallas as pl
from jax.experimental.pallas import tpu as pltpu

N_V = 2048
N_E = 4096
D = 256
TJ = 256


def _prep_body(hvt_ref, tt_ref, ef_ref, w_ref, p_ref, lsc_ref, ew_ref, hp_ref):
    i = pl.program_id(0)

    @pl.when(i == 0)
    def _():
        hp_ref[...] = jnp.dot(p_ref[...], hvt_ref[...],
                              preferred_element_type=jnp.float32)
        ew_ref[...] = jnp.dot(
            ef_ref[...].astype(jnp.bfloat16), w_ref[...].astype(jnp.bfloat16),
            preferred_element_type=jnp.float32).astype(jnp.bfloat16)

    lsc_ref[...] = (tt_ref[...] * hp_ref[...]).astype(jnp.bfloat16)


def _main_body(lsc_ref, t_ref, adj_ref, ew_ref, bias_ref, out_ref, acc_ref):
    j = pl.program_id(0)
    nj = pl.num_programs(0)

    mult = jnp.dot(lsc_ref[...], t_ref[...].astype(jnp.bfloat16),
                   preferred_element_type=jnp.float32)
    rows = jax.lax.broadcasted_iota(jnp.int32, (N_E, TJ), 0)
    cols = jax.lax.broadcasted_iota(jnp.int32, (N_E, TJ), 1) + j * TJ
    adj = adj_ref[...]
    a = jnp.where(rows == cols, adj, mult * adj)
    cmax = jnp.max(a, axis=0, keepdims=True)
    contrib = jnp.dot((a * (1.0 / cmax)).astype(jnp.bfloat16), ew_ref[...],
                      preferred_element_type=jnp.float32)

    @pl.when(j == 0)
    def _():
        acc_ref[...] = contrib

    @pl.when(j > 0)
    def _():
        acc_ref[...] += contrib

    @pl.when(j == nj - 1)
    def _():
        out_ref[...] = acc_ref[...] + bias_ref[...]


def kernel(H_v, edge_features, adj_e, adj_v, T, weight, bias, p):
    del adj_v
    t_t = T.T
    h_v_t = H_v.T

    n_prep = 4
    lsc, ew = pl.pallas_call(
        _prep_body,
        grid=(n_prep,),
        in_specs=[
            pl.BlockSpec((D, N_V), lambda i: (0, 0)),
            pl.BlockSpec((N_E // n_prep, N_V), lambda i: (i, 0)),
            pl.BlockSpec((N_E, D), lambda i: (0, 0)),
            pl.BlockSpec((D, D), lambda i: (0, 0)),
            pl.BlockSpec((1, D), lambda i: (0, 0)),
        ],
        out_specs=[
            pl.BlockSpec((N_E // n_prep, N_V), lambda i: (i, 0)),
            pl.BlockSpec((N_E, D), lambda i: (0, 0)),
        ],
        out_shape=[
            jax.ShapeDtypeStruct((N_E, N_V), jnp.bfloat16),
            jax.ShapeDtypeStruct((N_E, D), jnp.bfloat16),
        ],
        scratch_shapes=[pltpu.VMEM((1, N_V), jnp.float32)],
    )(h_v_t, t_t, edge_features, weight, p)

    out = pl.pallas_call(
        _main_body,
        grid=(N_E // TJ,),
        in_specs=[
            pl.BlockSpec((N_E, N_V), lambda j: (0, 0)),
            pl.BlockSpec((N_V, TJ), lambda j: (0, j)),
            pl.BlockSpec((N_E, TJ), lambda j: (0, j)),
            pl.BlockSpec((TJ, D), lambda j: (j, 0)),
            pl.BlockSpec((1, D), lambda j: (0, 0)),
        ],
        out_specs=pl.BlockSpec((N_E, D), lambda j: (0, 0)),
        out_shape=jax.ShapeDtypeStruct((N_E, D), jnp.float32),
        scratch_shapes=[pltpu.VMEM((N_E, D), jnp.float32)],
        compiler_params=pltpu.CompilerParams(
            vmem_limit_bytes=100 * 1024 * 1024),
    )(lsc, T, adj_e, ew, bias.reshape(1, D))

    return out


# speedup vs baseline: 1.2587x; 1.2001x over previous
import jax
import jax.numpy as jnp
from jax.experimental import pallas as pl
from jax.experimental.pallas import tpu as pltpu

N_V = 2048
N_E = 4096
D = 256
TJ = 256
TP = 1024


def _prep_body(hv_ref, t_ref, ef_ref, w_ref, p_ref, tsc_ref, ew_ref, hp_ref):
    i = pl.program_id(0)

    @pl.when(i == 0)
    def _():
        hp_ref[...] = jnp.dot(hv_ref[...], p_ref[...],
                              preferred_element_type=jnp.float32)
        ew_ref[...] = jnp.dot(
            ef_ref[...].astype(jnp.bfloat16), w_ref[...].astype(jnp.bfloat16),
            preferred_element_type=jnp.float32).astype(jnp.bfloat16)

    tsc_ref[...] = (t_ref[...] * hp_ref[...]).astype(jnp.bfloat16)


def _main_body(tsc_ref, t_ref, adj_ref, ew_ref, bias_ref, out_ref, acc_ref):
    j = pl.program_id(0)
    nj = pl.num_programs(0)

    mult = jax.lax.dot_general(
        tsc_ref[...], t_ref[...].astype(jnp.bfloat16),
        (((0,), (0,)), ((), ())),
        preferred_element_type=jnp.float32)

    rows = jax.lax.broadcasted_iota(jnp.int32, (N_E, TJ), 0)
    cols = jax.lax.broadcasted_iota(jnp.int32, (N_E, TJ), 1) + j * TJ
    adj = adj_ref[...]
    a = jnp.where(rows == cols, adj, mult * adj)
    cmax = jnp.max(a, axis=0, keepdims=True)
    a_n = (a * (1.0 / cmax)).astype(jnp.bfloat16)
    contrib = jnp.dot(a_n, ew_ref[...],
                      preferred_element_type=jnp.float32)

    @pl.when(j == 0)
    def _():
        acc_ref[...] = contrib

    @pl.when(j > 0)
    def _():
        acc_ref[...] += contrib

    @pl.when(j == nj - 1)
    def _():
        out_ref[...] = acc_ref[...] + bias_ref[...]


def kernel(H_v, edge_features, adj_e, adj_v, T, weight, bias, p):
    del adj_v

    tsc, ew = pl.pallas_call(
        _prep_body,
        grid=(N_E // TP,),
        in_specs=[
            pl.BlockSpec((N_V, D), lambda i: (0, 0)),
            pl.BlockSpec((N_V, TP), lambda i: (0, i)),
            pl.BlockSpec((N_E, D), lambda i: (0, 0)),
            pl.BlockSpec((D, D), lambda i: (0, 0)),
            pl.BlockSpec((D, 1), lambda i: (0, 0)),
        ],
        out_specs=[
            pl.BlockSpec((N_V, TP), lambda i: (0, i)),
            pl.BlockSpec((N_E, D), lambda i: (0, 0)),
        ],
        out_shape=[
            jax.ShapeDtypeStruct((N_V, N_E), jnp.bfloat16),
            jax.ShapeDtypeStruct((N_E, D), jnp.bfloat16),
        ],
        scratch_shapes=[pltpu.VMEM((N_V, 1), jnp.float32)],
    )(H_v, T, edge_features, weight, p.reshape(D, 1))

    return pl.pallas_call(
        _main_body,
        grid=(N_E // TJ,),
        in_specs=[
            pl.BlockSpec((N_V, N_E), lambda j: (0, 0)),
            pl.BlockSpec((N_V, TJ), lambda j: (0, j)),
            pl.BlockSpec((N_E, TJ), lambda j: (0, j)),
            pl.BlockSpec((TJ, D), lambda j: (j, 0)),
            pl.BlockSpec((1, D), lambda j: (0, 0)),
        ],
        out_specs=pl.BlockSpec((N_E, D), lambda j: (0, 0)),
        out_shape=jax.ShapeDtypeStruct((N_E, D), jnp.float32),
        scratch_shapes=[pltpu.VMEM((N_E, D), jnp.float32)],
        compiler_params=pltpu.CompilerParams(
            vmem_limit_bytes=63 * 1024 * 1024),
    )(tsc, T, adj_e, ew, bias.reshape(1, D))


# speedup vs baseline: 1.3378x; 1.0628x over previous
import jax
import jax.numpy as jnp
from jax.experimental import pallas as pl
from jax.experimental.pallas import tpu as pltpu

N_V = 2048
N_E = 4096
D = 256
TJ = 256
TP = 1024


def _prep_body(hv_ref, t_ref, ef_ref, w_ref, p_ref, tsc_ref, ew_ref, hp_ref):
    i = pl.program_id(0)

    @pl.when(i == 0)
    def _():
        hp_ref[...] = jnp.dot(hv_ref[...], p_ref[...],
                              preferred_element_type=jnp.float32)
        ew_ref[...] = jnp.dot(
            ef_ref[...].astype(jnp.bfloat16), w_ref[...].astype(jnp.bfloat16),
            preferred_element_type=jnp.float32).astype(jnp.bfloat16)

    tsc_ref[...] = jnp.transpose(
        (t_ref[...] * hp_ref[...]).astype(jnp.bfloat16))


def _main_body(tsc_ref, t_ref, adj_ref, ew_ref, bias_ref, out_ref, acc_ref):
    j = pl.program_id(0)
    nj = pl.num_programs(0)

    mult = jnp.dot(
        tsc_ref[...], t_ref[...].astype(jnp.bfloat16),
        preferred_element_type=jnp.float32)

    rows = jax.lax.broadcasted_iota(jnp.int32, (N_E, TJ), 0)
    cols = jax.lax.broadcasted_iota(jnp.int32, (N_E, TJ), 1) + j * TJ
    adj = adj_ref[...]
    a = jnp.where(rows == cols, adj, mult * adj)
    cmax = jnp.max(a, axis=0, keepdims=True)
    a_n = (a * (1.0 / cmax)).astype(jnp.bfloat16)
    contrib = jnp.dot(a_n, ew_ref[...],
                      preferred_element_type=jnp.float32)

    @pl.when(j == 0)
    def _():
        acc_ref[...] = contrib

    @pl.when(j > 0)
    def _():
        acc_ref[...] += contrib

    @pl.when(j == nj - 1)
    def _():
        out_ref[...] = acc_ref[...] + bias_ref[...]


def kernel(H_v, edge_features, adj_e, adj_v, T, weight, bias, p):
    del adj_v

    tsc, ew = pl.pallas_call(
        _prep_body,
        grid=(N_E // TP,),
        in_specs=[
            pl.BlockSpec((N_V, D), lambda i: (0, 0)),
            pl.BlockSpec((N_V, TP), lambda i: (0, i)),
            pl.BlockSpec((N_E, D), lambda i: (0, 0)),
            pl.BlockSpec((D, D), lambda i: (0, 0)),
            pl.BlockSpec((D, 1), lambda i: (0, 0)),
        ],
        out_specs=[
            pl.BlockSpec((TP, N_V), lambda i: (i, 0)),
            pl.BlockSpec((N_E, D), lambda i: (0, 0)),
        ],
        out_shape=[
            jax.ShapeDtypeStruct((N_E, N_V), jnp.bfloat16),
            jax.ShapeDtypeStruct((N_E, D), jnp.bfloat16),
        ],
        scratch_shapes=[pltpu.VMEM((N_V, 1), jnp.float32)],
    )(H_v, T, edge_features, weight, p.reshape(D, 1))

    return pl.pallas_call(
        _main_body,
        grid=(N_E // TJ,),
        in_specs=[
            pl.BlockSpec((N_E, N_V), lambda j: (0, 0)),
            pl.BlockSpec((N_V, TJ), lambda j: (0, j)),
            pl.BlockSpec((N_E, TJ), lambda j: (0, j)),
            pl.BlockSpec((TJ, D), lambda j: (j, 0)),
            pl.BlockSpec((1, D), lambda j: (0, 0)),
        ],
        out_specs=pl.BlockSpec((N_E, D), lambda j: (0, 0)),
        out_shape=jax.ShapeDtypeStruct((N_E, D), jnp.float32),
        scratch_shapes=[pltpu.VMEM((N_E, D), jnp.float32)],
        compiler_params=pltpu.CompilerParams(
            vmem_limit_bytes=63 * 1024 * 1024),
    )(tsc, T, adj_e, ew, bias.reshape(1, D))


# speedup vs baseline: 1.4406x; 1.0769x over previous
import jax
import jax.numpy as jnp
from jax.experimental import pallas as pl
from jax.experimental.pallas import tpu as pltpu

N_V = 2048
N_E = 4096
D = 256
TJ = 256
NJ = N_E // TJ
TP = 1024


def _prep_body(hv_ref, t_ref, ef_ref, w_ref, p_ref, tsc_ref, ew_ref, hp_ref):
    i = pl.program_id(0)

    @pl.when(i == 0)
    def _():
        hp_ref[...] = jnp.dot(hv_ref[...], p_ref[...],
                              preferred_element_type=jnp.float32)
        ew_ref[...] = jnp.dot(
            ef_ref[...].astype(jnp.bfloat16), w_ref[...].astype(jnp.bfloat16),
            preferred_element_type=jnp.float32).astype(jnp.bfloat16)

    tsc_ref[...] = jnp.transpose(
        (t_ref[...] * hp_ref[...]).astype(jnp.bfloat16))


def _main_body(tsc_ref, t_ref, adj_ref, ew_ref, bias_ref, out_ref, mult_ref):
    j = pl.program_id(0)

    @pl.when(j < NJ)
    def _():
        mult_ref[j % 2] = jnp.dot(
            tsc_ref[...], t_ref[...].astype(jnp.bfloat16),
            preferred_element_type=jnp.float32)

    @pl.when(j > 0)
    def _():
        rows = jax.lax.broadcasted_iota(jnp.int32, (N_E, TJ), 0)
        cols = jax.lax.broadcasted_iota(jnp.int32, (N_E, TJ), 1) + (j - 1) * TJ
        adj = adj_ref[...]
        a = jnp.where(rows == cols, adj, mult_ref[(j - 1) % 2] * adj)
        cmax = jnp.max(a, axis=0, keepdims=True)
        a_n = (a * (1.0 / cmax)).astype(jnp.bfloat16)
        contrib = jnp.dot(a_n, ew_ref[...],
                          preferred_element_type=jnp.float32)

        @pl.when(j == 1)
        def _():
            out_ref[...] = contrib + bias_ref[...]

        @pl.when(j > 1)
        def _():
            out_ref[...] += contrib


def kernel(H_v, edge_features, adj_e, adj_v, T, weight, bias, p):
    del adj_v

    tsc, ew = pl.pallas_call(
        _prep_body,
        grid=(N_E // TP,),
        in_specs=[
            pl.BlockSpec((N_V, D), lambda i: (0, 0)),
            pl.BlockSpec((N_V, TP), lambda i: (0, i)),
            pl.BlockSpec((N_E, D), lambda i: (0, 0)),
            pl.BlockSpec((D, D), lambda i: (0, 0)),
            pl.BlockSpec((D, 1), lambda i: (0, 0)),
        ],
        out_specs=[
            pl.BlockSpec((TP, N_V), lambda i: (i, 0)),
            pl.BlockSpec((N_E, D), lambda i: (0, 0)),
        ],
        out_shape=[
            jax.ShapeDtypeStruct((N_E, N_V), jnp.bfloat16),
            jax.ShapeDtypeStruct((N_E, D), jnp.bfloat16),
        ],
        scratch_shapes=[pltpu.VMEM((N_V, 1), jnp.float32)],
    )(H_v, T, edge_features, weight, p.reshape(D, 1))

    prev = lambda j: jnp.maximum(j - 1, 0)
    return pl.pallas_call(
        _main_body,
        grid=(NJ + 1,),
        in_specs=[
            pl.BlockSpec((N_E, N_V), lambda j: (0, 0)),
            pl.BlockSpec((N_V, TJ), lambda j: (0, jnp.minimum(j, NJ - 1))),
            pl.BlockSpec((N_E, TJ), lambda j: (0, prev(j))),
            pl.BlockSpec((TJ, D), lambda j: (prev(j), 0)),
            pl.BlockSpec((1, D), lambda j: (0, 0)),
        ],
        out_specs=pl.BlockSpec((N_E, D), lambda j: (0, 0)),
        out_shape=jax.ShapeDtypeStruct((N_E, D), jnp.float32),
        scratch_shapes=[pltpu.VMEM((2, N_E, TJ), jnp.float32)],
        compiler_params=pltpu.CompilerParams(
            vmem_limit_bytes=63 * 1024 * 1024),
    )(tsc, T, adj_e, ew, bias.reshape(1, D))


# speedup vs baseline: 1.4618x; 1.0148x over previous
import jax
import jax.numpy as jnp
from jax.experimental import pallas as pl
from jax.experimental.pallas import tpu as pltpu

N_V = 2048
N_E = 4096
D = 256
TJ = 256
NJ = N_E // TJ
TP = 1024


def _prep_body(hv_ref, t_ref, ef_ref, w_ref, p_ref, tsc_ref, ew_ref, hp_ref):
    i = pl.program_id(0)

    @pl.when(i == 0)
    def _():
        hp_ref[...] = jnp.dot(hv_ref[...], p_ref[...],
                              preferred_element_type=jnp.float32)
        ew_ref[...] = jnp.dot(
            ef_ref[...].astype(jnp.bfloat16), w_ref[...].astype(jnp.bfloat16),
            preferred_element_type=jnp.float32).astype(jnp.bfloat16)

    tsc_ref[...] = jnp.transpose(
        (t_ref[...] * hp_ref[...]).astype(jnp.bfloat16))


def _main_body(tsc_ref, t_ref, adj_ref, ew_ref, bias_ref, out_ref, mult_ref):
    j = pl.program_id(0)

    @pl.when(j < NJ)
    def _():
        mult_ref[j % 2] = jnp.dot(
            tsc_ref[...], t_ref[...].astype(jnp.bfloat16),
            preferred_element_type=jnp.float32).astype(jnp.bfloat16)

    @pl.when(j > 0)
    def _():
        r0 = (j - 1) * TJ
        eye = (jax.lax.broadcasted_iota(jnp.int32, (TJ, TJ), 0) ==
               jax.lax.broadcasted_iota(jnp.int32, (TJ, TJ), 1))
        sub = mult_ref[(j - 1) % 2, pl.ds(r0, TJ), :]
        mult_ref[(j - 1) % 2, pl.ds(r0, TJ), :] = jnp.where(
            eye, jnp.bfloat16(1.0), sub)
        a = mult_ref[(j - 1) % 2].astype(jnp.float32) * adj_ref[...]
        cmax = jnp.max(a, axis=0, keepdims=True)
        a_n = (a * (1.0 / cmax)).astype(jnp.bfloat16)
        contrib = jnp.dot(a_n, ew_ref[...],
                          preferred_element_type=jnp.float32)

        @pl.when(j == 1)
        def _():
            out_ref[...] = contrib + bias_ref[...]

        @pl.when(j > 1)
        def _():
            out_ref[...] += contrib


def kernel(H_v, edge_features, adj_e, adj_v, T, weight, bias, p):
    del adj_v

    tsc, ew = pl.pallas_call(
        _prep_body,
        grid=(N_E // TP,),
        in_specs=[
            pl.BlockSpec((N_V, D), lambda i: (0, 0)),
            pl.BlockSpec((N_V, TP), lambda i: (0, i)),
            pl.BlockSpec((N_E, D), lambda i: (0, 0)),
            pl.BlockSpec((D, D), lambda i: (0, 0)),
            pl.BlockSpec((D, 1), lambda i: (0, 0)),
        ],
        out_specs=[
            pl.BlockSpec((TP, N_V), lambda i: (i, 0)),
            pl.BlockSpec((N_E, D), lambda i: (0, 0)),
        ],
        out_shape=[
            jax.ShapeDtypeStruct((N_E, N_V), jnp.bfloat16),
            jax.ShapeDtypeStruct((N_E, D), jnp.bfloat16),
        ],
        scratch_shapes=[pltpu.VMEM((N_V, 1), jnp.float32)],
    )(H_v, T, edge_features, weight, p.reshape(D, 1))

    prev = lambda j: jnp.maximum(j - 1, 0)
    return pl.pallas_call(
        _main_body,
        grid=(NJ + 1,),
        in_specs=[
            pl.BlockSpec((N_E, N_V), lambda j: (0, 0)),
            pl.BlockSpec((N_V, TJ), lambda j: (0, jnp.minimum(j, NJ - 1))),
            pl.BlockSpec((N_E, TJ), lambda j: (0, prev(j))),
            pl.BlockSpec((TJ, D), lambda j: (prev(j), 0)),
            pl.BlockSpec((1, D), lambda j: (0, 0)),
        ],
        out_specs=pl.BlockSpec((N_E, D), lambda j: (0, 0)),
        out_shape=jax.ShapeDtypeStruct((N_E, D), jnp.float32),
        scratch_shapes=[pltpu.VMEM((2, N_E, TJ), jnp.bfloat16)],
        compiler_params=pltpu.CompilerParams(
            vmem_limit_bytes=63 * 1024 * 1024),
    )(tsc, T, adj_e, ew, bias.reshape(1, D))


# speedup vs baseline: 1.5067x; 1.0307x over previous
import jax
import jax.numpy as jnp
from jax.experimental import pallas as pl
from jax.experimental.pallas import tpu as pltpu

N_V = 2048
N_E = 4096
D = 256
TJ = 256
NJ = N_E // TJ
TP = 256
PA = N_E // TP


def _body(hv_ref, ef_ref, adj_ref, t1_ref, t2_ref, w_ref, bias_ref, p_ref,
          out_ref, hp_ref, tsc_ref, ew_ref, mult_ref):
    g = pl.program_id(0)

    @pl.when(g == 0)
    def _():
        hp_ref[...] = jnp.dot(hv_ref[...], p_ref[...],
                              preferred_element_type=jnp.float32)

    @pl.when(g < PA)
    def _():
        tsc_ref[pl.ds(g * TP, TP), :] = jnp.transpose(
            (t1_ref[...] * hp_ref[...]).astype(jnp.bfloat16))
        ew_ref[pl.ds(g * TP, TP), :] = jnp.dot(
            ef_ref[...].astype(jnp.bfloat16), w_ref[...].astype(jnp.bfloat16),
            preferred_element_type=jnp.float32).astype(jnp.bfloat16)

    @pl.when((g >= PA) & (g < PA + NJ))
    def _():
        mult_ref[(g - PA) % 2] = jnp.dot(
            tsc_ref[...], t2_ref[...].astype(jnp.bfloat16),
            preferred_element_type=jnp.float32).astype(jnp.bfloat16)

    @pl.when(g > PA)
    def _():
        jj = g - PA - 1
        r0 = jj * TJ
        eye = (jax.lax.broadcasted_iota(jnp.int32, (TJ, TJ), 0) ==
               jax.lax.broadcasted_iota(jnp.int32, (TJ, TJ), 1))
        sub = mult_ref[jj % 2, pl.ds(r0, TJ), :]
        mult_ref[jj % 2, pl.ds(r0, TJ), :] = jnp.where(
            eye, jnp.bfloat16(1.0), sub)
        a = mult_ref[jj % 2].astype(jnp.float32) * adj_ref[...]
        cmax = jnp.max(a, axis=0, keepdims=True)
        a_n = (a * (1.0 / cmax)).astype(jnp.bfloat16)
        contrib = jnp.dot(a_n, ew_ref[pl.ds(r0, TJ), :],
                          preferred_element_type=jnp.float32)

        @pl.when(g == PA + 1)
        def _():
            out_ref[...] = contrib + bias_ref[...]

        @pl.when(g > PA + 1)
        def _():
            out_ref[...] += contrib


def kernel(H_v, edge_features, adj_e, adj_v, T, weight, bias, p):
    del adj_v

    clip = lambda v, hi: jnp.clip(v, 0, hi)
    return pl.pallas_call(
        _body,
        grid=(PA + NJ + 1,),
        in_specs=[
            pl.BlockSpec((N_V, D), lambda g: (0, 0)),
            pl.BlockSpec((TP, D), lambda g: (clip(g, PA - 1), 0)),
            pl.BlockSpec((N_E, TJ),
                         lambda g: (0, clip(g - PA - 1, NJ - 1))),
            pl.BlockSpec((N_V, TP), lambda g: (0, clip(g, PA - 1))),
            pl.BlockSpec((N_V, TJ),
                         lambda g: (0, clip(g - PA, NJ - 1))),
            pl.BlockSpec((D, D), lambda g: (0, 0)),
            pl.BlockSpec((1, D), lambda g: (0, 0)),
            pl.BlockSpec((D, 1), lambda g: (0, 0)),
        ],
        out_specs=pl.BlockSpec((N_E, D), lambda g: (0, 0)),
        out_shape=jax.ShapeDtypeStruct((N_E, D), jnp.float32),
        scratch_shapes=[
            pltpu.VMEM((N_V, 1), jnp.float32),
            pltpu.VMEM((N_E, N_V), jnp.bfloat16),
            pltpu.VMEM((N_E, D), jnp.bfloat16),
            pltpu.VMEM((2, N_E, TJ), jnp.bfloat16),
        ],
        compiler_params=pltpu.CompilerParams(
            vmem_limit_bytes=63 * 1024 * 1024),
    )(H_v, edge_features, adj_e, T, T, weight, bias.reshape(1, D),
      p.reshape(D, 1))


# speedup vs baseline: 1.5294x; 1.0151x over previous
import jax
import jax.numpy as jnp
from jax.experimental import pallas as pl
from jax.experimental.pallas import tpu as pltpu

N_V = 2048
N_E = 4096
D = 256
TJ = 256
NJ = N_E // TJ
PA = NJ


def _body(hv_ref, ef_ref, adj_ref, t_ref, w_ref, bias_ref, p_ref,
          out_ref, hp_ref, tsc_ref, ew_ref, mult_ref):
    g = pl.program_id(0)

    @pl.when(g == 0)
    def _():
        hp_ref[...] = jnp.dot(hv_ref[...], p_ref[...],
                              preferred_element_type=jnp.float32)

    @pl.when(g < PA)
    def _():
        tsc_ref[pl.ds(g * TJ, TJ), :] = jnp.transpose(
            (t_ref[...] * hp_ref[...]).astype(jnp.bfloat16))
        ew_ref[pl.ds(g * TJ, TJ), :] = jnp.dot(
            ef_ref[...].astype(jnp.bfloat16), w_ref[...].astype(jnp.bfloat16),
            preferred_element_type=jnp.float32).astype(jnp.bfloat16)

    @pl.when(g > PA)
    def _():
        s = g - PA - 1
        half = (s % 2) * TJ
        r0 = s * TJ
        eye = (jax.lax.broadcasted_iota(jnp.int32, (TJ, TJ), 0) ==
               jax.lax.broadcasted_iota(jnp.int32, (TJ, TJ), 1))
        sub = mult_ref[pl.ds(r0, TJ), pl.ds(half, TJ)]
        mult_ref[pl.ds(r0, TJ), pl.ds(half, TJ)] = jnp.where(
            eye, jnp.bfloat16(1.0), sub)
        a = (mult_ref[:, pl.ds(half, TJ)].astype(jnp.float32) *
             adj_ref[...])
        cmax = jnp.max(a, axis=0, keepdims=True)
        mult_ref[:, pl.ds(half, TJ)] = (
            a * (1.0 / cmax)).astype(jnp.bfloat16)

        @pl.when(s % 2 == 1)
        def _():
            contrib = jnp.dot(mult_ref[...],
                              ew_ref[pl.ds((s - 1) * TJ, 2 * TJ), :],
                              preferred_element_type=jnp.float32)

            @pl.when(s == 1)
            def _():
                out_ref[...] = contrib + bias_ref[...]

            @pl.when(s > 1)
            def _():
                out_ref[...] += contrib

    @pl.when((g >= PA) & (g < PA + NJ))
    def _():
        ss = g - PA
        mult_ref[:, pl.ds((ss % 2) * TJ, TJ)] = jnp.dot(
            tsc_ref[...], t_ref[...].astype(jnp.bfloat16),
            preferred_element_type=jnp.float32).astype(jnp.bfloat16)


def kernel(H_v, edge_features, adj_e, adj_v, T, weight, bias, p):
    del adj_v

    clip = lambda v, hi: jnp.clip(v, 0, hi)
    return pl.pallas_call(
        _body,
        grid=(PA + NJ + 1,),
        in_specs=[
            pl.BlockSpec((N_V, D), lambda g: (0, 0)),
            pl.BlockSpec((TJ, D), lambda g: (clip(g, PA - 1), 0)),
            pl.BlockSpec((N_E, TJ),
                         lambda g: (0, clip(g - PA - 1, NJ - 1))),
            pl.BlockSpec((N_V, TJ),
                         lambda g: (0, jnp.where(g < PA, g,
                                                 clip(g - PA, NJ - 1)))),
            pl.BlockSpec((D, D), lambda g: (0, 0)),
            pl.BlockSpec((1, D), lambda g: (0, 0)),
            pl.BlockSpec((D, 1), lambda g: (0, 0)),
        ],
        out_specs=pl.BlockSpec((N_E, D), lambda g: (0, 0)),
        out_shape=jax.ShapeDtypeStruct((N_E, D), jnp.float32),
        scratch_shapes=[
            pltpu.VMEM((N_V, 1), jnp.float32),
            pltpu.VMEM((N_E, N_V), jnp.bfloat16),
            pltpu.VMEM((N_E, D), jnp.bfloat16),
            pltpu.VMEM((N_E, 2 * TJ), jnp.bfloat16),
        ],
        compiler_params=pltpu.CompilerParams(
            vmem_limit_bytes=63 * 1024 * 1024),
    )(H_v, edge_features, adj_e, T, weight, bias.reshape(1, D),
      p.reshape(D, 1))
